# baseline (speedup 1.0000x reference)
import jax
import jax.numpy as jnp
from jax import lax
from jax.experimental import pallas as pl
from jax.experimental.pallas import tpu as pltpu
from jax.experimental.pallas import tpu_sc as plsc

_CHUNK_ROWS = 16
_NBUF = 3
_PREFETCH = 2


def kernel(input_ids, positional_encoding_table):
    seq_len = input_ids.shape[1]
    model_dim = positional_encoding_table.shape[1]

    info = plsc.get_sparse_core_info()
    nc, ns = info.num_cores, info.num_subcores
    nw = nc * ns
    rows_per_w = seq_len // nw
    assert rows_per_w * nw == seq_len
    nch = rows_per_w // _CHUNK_ROWS
    assert nch * _CHUNK_ROWS == rows_per_w and nch >= _NBUF

    mesh = plsc.VectorSubcoreMesh(core_axis_name="c", subcore_axis_name="s")

    @jax.jit
    def run(table):
        def body(table_hbm, out_hbm, buf, in_sems, out_sems):
            wid = lax.axis_index("s") * nc + lax.axis_index("c")
            base = wid * rows_per_w

            def in_copy(i, slot):
                return pltpu.make_async_copy(
                    table_hbm.at[pl.ds(base + i * _CHUNK_ROWS, _CHUNK_ROWS), :],
                    buf.at[slot],
                    in_sems.at[slot],
                )

            def out_copy(i, slot):
                return pltpu.make_async_copy(
                    buf.at[slot],
                    out_hbm.at[pl.ds(base + i * _CHUNK_ROWS, _CHUNK_ROWS), :],
                    out_sems.at[slot],
                )

            for b in range(_PREFETCH):
                in_copy(b, b).start()

            def step(i, _):
                slot = lax.rem(i, _NBUF)
                in_copy(i, slot).wait()
                out_copy(i, slot).start()
                j = i + _PREFETCH

                @pl.when(j < nch)
                def _():
                    jslot = lax.rem(j, _NBUF)

                    @pl.when(j >= _NBUF)
                    def _():
                        out_copy(j - _NBUF, jslot).wait()

                    in_copy(j, jslot).start()

                return 0

            lax.fori_loop(0, nch, step, 0)
            for k in range(nch - _NBUF, nch):
                out_copy(k, k % _NBUF).wait()

        return pl.kernel(
            body,
            out_type=jax.ShapeDtypeStruct((seq_len, model_dim), table.dtype),
            mesh=mesh,
            scratch_types=[
                pltpu.VMEM((_NBUF, _CHUNK_ROWS, model_dim), table.dtype),
                pltpu.SemaphoreType.DMA((_NBUF,)),
                pltpu.SemaphoreType.DMA((_NBUF,)),
            ],
        )(table)

    return run(positional_encoding_table)


# speedup vs baseline: 1.5748x; 1.5748x over previous
import jax
import jax.numpy as jnp
from jax.experimental import pallas as pl
from jax.experimental.pallas import tpu as pltpu

_BLOCK_ROWS = 1024


def kernel(input_ids, positional_encoding_table):
    seq_len = input_ids.shape[1]
    model_dim = positional_encoding_table.shape[1]
    grid = (seq_len // _BLOCK_ROWS,)

    def body(table_ref, out_ref):
        out_ref[...] = table_ref[...]

    return pl.pallas_call(
        body,
        out_shape=jax.ShapeDtypeStruct((seq_len, model_dim),
                                       positional_encoding_table.dtype),
        grid=grid,
        in_specs=[pl.BlockSpec((_BLOCK_ROWS, model_dim), lambda i: (i, 0))],
        out_specs=pl.BlockSpec((_BLOCK_ROWS, model_dim), lambda i: (i, 0)),
    )(positional_encoding_table)


# speedup vs baseline: 1.5764x; 1.0010x over previous
import jax
import jax.numpy as jnp
from jax.experimental import pallas as pl
from jax.experimental.pallas import tpu as pltpu

_BLOCK_ROWS = 1024


def kernel(input_ids, positional_encoding_table):
    seq_len = input_ids.shape[1]
    model_dim = positional_encoding_table.shape[1]
    block_rows = next(b for b in (_BLOCK_ROWS, 512, 256, 128, 64, 32, 16, 8,
                                  4, 2, 1) if seq_len % b == 0)
    grid = (seq_len // block_rows,)

    def body(table_ref, out_ref):
        out_ref[...] = table_ref[...]

    return pl.pallas_call(
        body,
        out_shape=jax.ShapeDtypeStruct((seq_len, model_dim),
                                       positional_encoding_table.dtype),
        grid=grid,
        in_specs=[pl.BlockSpec((block_rows, model_dim), lambda i: (i, 0))],
        out_specs=pl.BlockSpec((block_rows, model_dim), lambda i: (i, 0)),
    )(positional_encoding_table)
